# Initial kernel scaffold; baseline (speedup 1.0000x reference)
import jax
import jax.numpy as jnp
from jax.experimental import pallas as pl


def kernel(mol_x, mol_edge_index, mol_batch, protein_x, protein_edge_index, batch_size, Wm, bm, Wp, bp, W1, b1, W2, b2, W3, b3):
    raise NotImplementedError("write your pallas kernel here")


# speedup vs baseline: 9.6335x; 9.6335x over previous
import functools

import jax
import jax.numpy as jnp
from jax import lax
from jax.experimental import pallas as pl
from jax.experimental.pallas import tpu as pltpu
from jax.experimental.pallas import tpu_sc as plsc

HID = 128
NPROT = 5000
G = 4
R = 10368
DUMMY = R - 1
NTILES = 16
CHUNK = R // NTILES
EBATCH = 128
NB = 79
PER_TILE = NB * EBATCH
E_PAD = NTILES * PER_TILE
BR = 256


def _build_indices(mol_edge_index, mol_batch, protein_edge_index, batch_size):
    i32 = jnp.int32
    group = jnp.minimum(mol_batch, batch_size - 1).astype(i32)
    n_mol = group.shape[0]
    grp = jnp.arange(G, dtype=i32)
    cnt = jnp.sum((group[:, None] == grp[None, :]).astype(i32), axis=0)
    end = jnp.cumsum(cnt)
    start = end - cnt
    B1 = end[1] + 2 * NPROT

    def to_flat(p):
        own = (p >= B1).astype(i32)
        return p - own * B1 + own * R

    nodes = jnp.arange(n_mol, dtype=i32)
    mol_pos = nodes + group * NPROT
    ie = group[mol_edge_index[0]]
    src_m = mol_edge_index[0].astype(i32) + ie * NPROT
    dst_m = jnp.clip(mol_edge_index[1].astype(i32) - start[ie], 0,
                     cnt[ie] + NPROT - 1) + start[ie] + ie * NPROT
    center = end[group] + NPROT // 2 + group * NPROT

    S_sm = jnp.concatenate([src_m, mol_pos, center])
    D_sm = jnp.concatenate([dst_m, center, mol_pos])

    pe0 = protein_edge_index[0].astype(i32)
    pe1 = protein_edge_index[1].astype(i32)
    Ep = pe0.shape[0]
    E_SC = S_sm.shape[0] + 2 * Ep
    pad = E_PAD - E_SC

    src_sc, dst_sc = [], []
    for s in (0, 1):
        keep = (D_sm >= B1).astype(i32) == s
        dstloc = jnp.where(keep, D_sm - s * B1, DUMMY)
        srcs = [to_flat(S_sm)]
        dsts = [dstloc]
        for g in (2 * s, 2 * s + 1):
            off = end[g] + g * NPROT
            srcs.append(to_flat(pe0 + off))
            dsts.append(pe1 + off - s * B1)
        src_sc.append(jnp.concatenate(srcs + [jnp.zeros((pad,), i32)]))
        dst_sc.append(jnp.concatenate(dsts + [jnp.full((pad,), DUMMY, i32)]))
    src_all = jnp.concatenate(src_sc)
    dst_all = jnp.concatenate(dst_sc)

    flat_mol = to_flat(mol_pos)
    prot_starts = jnp.stack([to_flat(end[g] + g * NPROT) for g in range(G)])
    gs = jnp.stack([to_flat(start[g] + g * NPROT) for g in range(G)])
    ge = gs + cnt + NPROT
    counts = (cnt + NPROT).astype(jnp.float32)
    return src_all, dst_all, flat_mol, prot_starts, gs, ge, counts


@functools.cache
def _get_sc_scatter():
    mesh = plsc.VectorSubcoreMesh(core_axis_name="c", subcore_axis_name="s")

    @functools.partial(
        pl.kernel,
        mesh=mesh,
        out_type=jax.ShapeDtypeStruct((2 * R, HID), jnp.float32),
        scratch_types=[
            pltpu.VMEM((EBATCH,), jnp.int32),
            pltpu.VMEM((EBATCH,), jnp.int32),
            pltpu.VMEM((EBATCH, HID), jnp.float32),
            pltpu.VMEM_SHARED((R, HID), jnp.float32),
            pltpu.SemaphoreType.DMA,
        ],
    )
    def _sc_scatter(x_hbm, src_hbm, dst_hbm, zeros_hbm, out_hbm,
                    sidx, didx, rows, acc, sem):
        c = lax.axis_index("c")
        t = lax.axis_index("s")
        pltpu.sync_copy(zeros_hbm, acc.at[pl.ds(t * CHUNK, CHUNK)])
        plsc.subcore_barrier()

        base0 = c * E_PAD + t * PER_TILE

        def body(i, carry):
            base = base0 + i * EBATCH
            pltpu.sync_copy(src_hbm.at[pl.ds(base, EBATCH)], sidx)
            pltpu.sync_copy(dst_hbm.at[pl.ds(base, EBATCH)], didx)
            pltpu.async_copy(x_hbm.at[sidx], rows, sem).wait()
            pltpu.sync_copy(rows, acc.at[didx], add=True)
            return carry

        lax.fori_loop(0, NB, body, 0)
        plsc.subcore_barrier()
        pltpu.sync_copy(acc.at[pl.ds(t * CHUNK, CHUNK)],
                        out_hbm.at[pl.ds(c * R + t * CHUNK, CHUNK)])

    return _sc_scatter


def _mm_bias_body(x_ref, w_ref, b_ref, o_ref):
    o_ref[...] = jnp.dot(x_ref[...], w_ref[...],
                         preferred_element_type=jnp.float32) + b_ref[...]


def _mm_bias(x, w, b, br):
    n = x.shape[0]
    return pl.pallas_call(
        _mm_bias_body,
        grid=(n // br,),
        in_specs=[pl.BlockSpec((br, x.shape[1]), lambda i: (i, 0)),
                  pl.BlockSpec((x.shape[1], HID), lambda i: (0, 0)),
                  pl.BlockSpec((1, HID), lambda i: (0, 0))],
        out_specs=pl.BlockSpec((br, HID), lambda i: (i, 0)),
        out_shape=jax.ShapeDtypeStruct((n, HID), jnp.float32),
    )(x, w, b)


def _mm_scale_body(x_ref, w_ref, deg_ref, o_ref):
    dinv = lax.rsqrt(deg_ref[...] + 1.0)
    o_ref[...] = dinv * jnp.dot(x_ref[...], w_ref[...],
                                preferred_element_type=jnp.float32)


def _mm_scale(x, w, degf):
    return pl.pallas_call(
        _mm_scale_body,
        grid=(2 * R // BR,),
        in_specs=[pl.BlockSpec((BR, HID), lambda i: (i, 0)),
                  pl.BlockSpec((HID, HID), lambda i: (0, 0)),
                  pl.BlockSpec((BR, HID), lambda i: (i, 0))],
        out_specs=pl.BlockSpec((BR, HID), lambda i: (i, 0)),
        out_shape=jax.ShapeDtypeStruct((2 * R, HID), jnp.float32),
    )(x, w, degf)


def _combine_body(s_ref, ht_ref, deg_ref, b_ref, w_ref, o_ref):
    dinv = lax.rsqrt(deg_ref[...] + 1.0)
    x = jnp.maximum(dinv * (s_ref[...] + ht_ref[...]) + b_ref[...], 0.0)
    o_ref[...] = dinv * jnp.dot(x, w_ref[...],
                                preferred_element_type=jnp.float32)


def _combine(s, ht, degf, b, w_next):
    return pl.pallas_call(
        _combine_body,
        grid=(2 * R // BR,),
        in_specs=[pl.BlockSpec((BR, HID), lambda i: (i, 0)),
                  pl.BlockSpec((BR, HID), lambda i: (i, 0)),
                  pl.BlockSpec((BR, HID), lambda i: (i, 0)),
                  pl.BlockSpec((1, HID), lambda i: (0, 0)),
                  pl.BlockSpec((HID, HID), lambda i: (0, 0))],
        out_specs=pl.BlockSpec((BR, HID), lambda i: (i, 0)),
        out_shape=jax.ShapeDtypeStruct((2 * R, HID), jnp.float32),
    )(s, ht, degf, b, w_next)


def _pool_body(m_ref, s_ref, ht_ref, deg_ref, b_ref, o_ref):
    i = pl.program_id(0)
    dinv = lax.rsqrt(deg_ref[...] + 1.0)
    y = dinv * (s_ref[...] + ht_ref[...]) + b_ref[...]

    @pl.when(i == 0)
    def _():
        o_ref[...] = jnp.zeros_like(o_ref)

    o_ref[...] += jnp.dot(m_ref[...], y, preferred_element_type=jnp.float32)


def _pool(s, ht, degf, b, mask):
    return pl.pallas_call(
        _pool_body,
        grid=(2 * R // BR,),
        in_specs=[pl.BlockSpec((8, BR), lambda i: (0, i)),
                  pl.BlockSpec((BR, HID), lambda i: (i, 0)),
                  pl.BlockSpec((BR, HID), lambda i: (i, 0)),
                  pl.BlockSpec((BR, HID), lambda i: (i, 0)),
                  pl.BlockSpec((1, HID), lambda i: (0, 0))],
        out_specs=pl.BlockSpec((8, HID), lambda i: (0, 0)),
        out_shape=jax.ShapeDtypeStruct((8, HID), jnp.float32),
    )(mask, s, ht, degf, b)


def kernel(mol_x, mol_edge_index, mol_batch, protein_x, protein_edge_index,
           batch_size, Wm, bm, Wp, bp, W1, b1, W2, b2, W3, b3):
    f32 = jnp.float32
    src_all, dst_all, flat_mol, prot_starts, gs, ge, counts = _build_indices(
        mol_edge_index, mol_batch, protein_edge_index, batch_size)

    sc_scatter = _get_sc_scatter()
    zeros_chunk = jnp.zeros((CHUNK, HID), f32)
    ones_tab = jnp.ones((2 * R, HID), f32)

    degf = sc_scatter(ones_tab, src_all, dst_all, zeros_chunk)

    mol_xp = jnp.pad(mol_x.astype(f32), ((0, 0), (0, HID - mol_x.shape[1])))
    Wmp = jnp.pad(Wm.astype(f32), ((0, HID - Wm.shape[0]), (0, 0)))
    prot_xp = jnp.pad(protein_x.astype(f32),
                      ((0, 120), (0, HID - protein_x.shape[1])))
    Wpp = jnp.pad(Wp.astype(f32), ((0, HID - Wp.shape[0]), (0, 0)))
    mol_feats = _mm_bias(mol_xp, Wmp, bm.reshape(1, HID), 320)
    prot_feats = _mm_bias(prot_xp, Wpp, bp.reshape(1, HID), 640)[:NPROT]

    x0 = jnp.zeros((2 * R, HID), f32)
    x0 = x0.at[flat_mol].set(mol_feats)
    for g in range(G):
        x0 = lax.dynamic_update_slice(x0, prot_feats, (prot_starts[g], 0))

    ht = _mm_scale(x0, W1.astype(f32), degf)
    s1 = sc_scatter(ht, src_all, dst_all, zeros_chunk)
    ht = _combine(s1, ht, degf, b1.reshape(1, HID), W2.astype(f32))
    s2 = sc_scatter(ht, src_all, dst_all, zeros_chunk)
    ht = _combine(s2, ht, degf, b2.reshape(1, HID), W3.astype(f32))
    s3 = sc_scatter(ht, src_all, dst_all, zeros_chunk)

    rows = jnp.arange(2 * R, dtype=jnp.int32)
    mask = ((rows[None, :] >= gs[:, None]) &
            (rows[None, :] < ge[:, None])).astype(f32)
    mask = jnp.concatenate([mask, jnp.zeros((8 - G, 2 * R), f32)])
    sums = _pool(s3, ht, degf, b3.reshape(1, HID), mask)[:G]
    return sums / counts[:, None]
